# baseline (speedup 1.0000x reference)
import functools

import jax
import jax.numpy as jnp
from jax.experimental import pallas as pl
from jax.experimental.pallas import tpu as pltpu

N = 10000
M = 2048
D_IN = 256
D_OUT = 256
TN = 1000


def _phase1_kernel(x_ref, h_ref, w_ref, b_ref, he_ref):
    i = pl.program_id(0)
    xl = jax.lax.dot_general(
        x_ref[...], w_ref[...],
        dimension_numbers=(((1,), (1,)), ((), ())),
        preferred_element_type=jnp.float32,
    ) + b_ref[...]
    acc = jax.lax.dot_general(
        h_ref[...].astype(jnp.bfloat16), xl.astype(jnp.bfloat16),
        dimension_numbers=(((0,), (0,)), ((), ())),
        preferred_element_type=jnp.float32,
    )

    @pl.when(i == 0)
    def _():
        he_ref[...] = jnp.zeros_like(he_ref)

    he_ref[...] += acc


def _phase2_kernel(h_ref, he_ref, o_ref):
    o_ref[...] = jax.lax.dot_general(
        h_ref[...].astype(jnp.bfloat16), he_ref[...],
        dimension_numbers=(((1,), (0,)), ((), ())),
        preferred_element_type=jnp.float32,
    )


@functools.partial(jax.jit, static_argnames=())
def kernel(X, H_sparse, W, b):
    nb = N // TN
    b2 = b.reshape(1, D_OUT)

    he = pl.pallas_call(
        _phase1_kernel,
        grid=(nb,),
        in_specs=[
            pl.BlockSpec((TN, D_IN), lambda i: (i, 0)),
            pl.BlockSpec((TN, M), lambda i: (i, 0)),
            pl.BlockSpec((D_OUT, D_IN), lambda i: (0, 0)),
            pl.BlockSpec((1, D_OUT), lambda i: (0, 0)),
        ],
        out_specs=pl.BlockSpec((M, D_OUT), lambda i: (0, 0)),
        out_shape=jax.ShapeDtypeStruct((M, D_OUT), jnp.float32),
        compiler_params=pltpu.CompilerParams(
            dimension_semantics=("arbitrary",),
        ),
    )(X, H_sparse, W, b2)

    he_b = he.astype(jnp.bfloat16)

    out = pl.pallas_call(
        _phase2_kernel,
        grid=(nb,),
        in_specs=[
            pl.BlockSpec((TN, M), lambda i: (i, 0)),
            pl.BlockSpec((M, D_OUT), lambda i: (0, 0)),
        ],
        out_specs=pl.BlockSpec((TN, D_OUT), lambda i: (i, 0)),
        out_shape=jax.ShapeDtypeStruct((N, D_OUT), jnp.float32),
        compiler_params=pltpu.CompilerParams(
            dimension_semantics=("parallel",),
        ),
    )(H_sparse, he_b)

    return out


# speedup vs baseline: 1.0480x; 1.0480x over previous
import functools

import jax
import jax.numpy as jnp
from jax.experimental import pallas as pl
from jax.experimental.pallas import tpu as pltpu

N = 10000
M = 2048
D_IN = 256
D_OUT = 256
TN = 400
NB = N // TN


def _fused_kernel(x_ref, h_ref, w_ref, b_ref, o_ref, hb_ref, he_ref):
    i = pl.program_id(0)

    @pl.when(i < NB)
    def _phase1():
        hb = h_ref[...].astype(jnp.bfloat16)
        hb_ref[i] = hb
        xl = jax.lax.dot_general(
            x_ref[...], w_ref[...],
            dimension_numbers=(((1,), (1,)), ((), ())),
            preferred_element_type=jnp.float32,
        ) + b_ref[...]
        acc = jax.lax.dot_general(
            hb, xl.astype(jnp.bfloat16),
            dimension_numbers=(((0,), (0,)), ((), ())),
            preferred_element_type=jnp.float32,
        )

        @pl.when(i == 0)
        def _():
            he_ref[...] = jnp.zeros_like(he_ref)

        he_ref[...] += acc

    @pl.when(i >= NB)
    def _phase2():
        j = i - NB
        o_ref[...] = jax.lax.dot_general(
            hb_ref[j], he_ref[...].astype(jnp.bfloat16),
            dimension_numbers=(((1,), (0,)), ((), ())),
            preferred_element_type=jnp.float32,
        )


@functools.partial(jax.jit, static_argnames=())
def kernel(X, H_sparse, W, b):
    b2 = b.reshape(1, D_OUT)

    out = pl.pallas_call(
        _fused_kernel,
        grid=(2 * NB,),
        in_specs=[
            pl.BlockSpec((TN, D_IN), lambda i: (jnp.minimum(i, NB - 1), 0)),
            pl.BlockSpec((TN, M), lambda i: (jnp.minimum(i, NB - 1), 0)),
            pl.BlockSpec((D_OUT, D_IN), lambda i: (0, 0)),
            pl.BlockSpec((1, D_OUT), lambda i: (0, 0)),
        ],
        out_specs=pl.BlockSpec((TN, D_OUT), lambda i: (jnp.maximum(i - NB, 0), 0)),
        out_shape=jax.ShapeDtypeStruct((N, D_OUT), jnp.float32),
        scratch_shapes=[
            pltpu.VMEM((NB, TN, M), jnp.bfloat16),
            pltpu.VMEM((M, D_OUT), jnp.float32),
        ],
        compiler_params=pltpu.CompilerParams(
            dimension_semantics=("arbitrary",),
        ),
    )(X, H_sparse, W, b2)

    return out


# speedup vs baseline: 1.0763x; 1.0270x over previous
import functools

import jax
import jax.numpy as jnp
from jax.experimental import pallas as pl
from jax.experimental.pallas import tpu as pltpu

N = 10000
M = 2048
D_IN = 256
D_OUT = 256
TN = 400
NB = N // TN


def _fused_kernel(x_hbm, h_hbm, w_ref, b_ref, o_hbm, hb_ref, he_ref):
    he_ref[...] = jnp.zeros_like(he_ref)

    def p1_body(idx, x_vmem, h_vmem):
        (i,) = idx
        hb = h_vmem[...].astype(jnp.bfloat16)
        hb_ref[i] = hb
        xl = jax.lax.dot_general(
            x_vmem[...], w_ref[...],
            dimension_numbers=(((1,), (1,)), ((), ())),
            preferred_element_type=jnp.float32,
        ) + b_ref[...]
        he_ref[...] += jax.lax.dot_general(
            hb, xl.astype(jnp.bfloat16),
            dimension_numbers=(((0,), (0,)), ((), ())),
            preferred_element_type=jnp.float32,
        )

    pltpu.emit_pipeline(
        p1_body,
        grid=(NB,),
        in_specs=[
            pl.BlockSpec((TN, D_IN), lambda i: (i, 0)),
            pl.BlockSpec((TN, M), lambda i: (i, 0)),
        ],
        _explicit_indices=True,
    )(x_hbm, h_hbm)

    def p2_body(idx, o_vmem):
        (j,) = idx
        o_vmem[...] = jax.lax.dot_general(
            hb_ref[j], he_ref[...].astype(jnp.bfloat16),
            dimension_numbers=(((1,), (0,)), ((), ())),
            preferred_element_type=jnp.float32,
        )

    pltpu.emit_pipeline(
        p2_body,
        grid=(NB,),
        out_specs=[pl.BlockSpec((TN, D_OUT), lambda j: (j, 0))],
        _explicit_indices=True,
    )(o_hbm)


@functools.partial(jax.jit, static_argnames=())
def kernel(X, H_sparse, W, b):
    b2 = b.reshape(1, D_OUT)

    out = pl.pallas_call(
        _fused_kernel,
        in_specs=[
            pl.BlockSpec(memory_space=pltpu.MemorySpace.HBM),
            pl.BlockSpec(memory_space=pltpu.MemorySpace.HBM),
            pl.BlockSpec(memory_space=pltpu.MemorySpace.VMEM),
            pl.BlockSpec(memory_space=pltpu.MemorySpace.VMEM),
        ],
        out_specs=pl.BlockSpec(memory_space=pltpu.MemorySpace.HBM),
        out_shape=jax.ShapeDtypeStruct((N, D_OUT), jnp.float32),
        scratch_shapes=[
            pltpu.VMEM((NB, TN, M), jnp.bfloat16),
            pltpu.VMEM((M, D_OUT), jnp.float32),
        ],
    )(X, H_sparse, W, b2)

    return out
